# baseline (speedup 1.0000x reference)
import functools

import jax
import jax.numpy as jnp
from jax import lax
from jax.experimental import pallas as pl
from jax.experimental.pallas import tpu as pltpu
from jax.experimental.pallas import tpu_sc as plsc

_B = 1024
_M = 16
_C = 128
_NNZ = 512
_NT = 34

_LANES = 16
_NW = 32
_BPW = _B // _NW
_CCHUNKS = _C // _LANES
_TPTR_PAD = 48


def _sc_tensor_product(x1, x2, w, cg, pidx, tptr_pad):
    mesh = plsc.VectorSubcoreMesh(core_axis_name="c", subcore_axis_name="s")

    @functools.partial(
        pl.kernel,
        mesh=mesh,
        out_type=jax.ShapeDtypeStruct((_B, _M, _C), jnp.float32),
        compiler_params=pltpu.CompilerParams(needs_layout_passes=False),
        scratch_types=[
            pltpu.SMEM((_NNZ,), jnp.int32),
            pltpu.SMEM((_NNZ,), jnp.float32),
            pltpu.SMEM((_TPTR_PAD,), jnp.int32),
            pltpu.VMEM((_NNZ,), jnp.int32),
            pltpu.VMEM((_NNZ,), jnp.float32),
            pltpu.VMEM((_TPTR_PAD,), jnp.int32),
            pltpu.VMEM((_M, _C), jnp.float32),
            pltpu.VMEM((_M, _C), jnp.float32),
            pltpu.VMEM((_NT, _C), jnp.float32),
            pltpu.VMEM((_M, _C), jnp.float32),
        ],
    )
    def k(x1_hbm, x2_hbm, w_hbm, cg_hbm, pidx_hbm, tptr_hbm,
          out_hbm, pidx_s, cg_s, tptr_s, pidx_b, cg_b, tptr_b,
          x1_v, x2_v, w_v, out_v):
        wid = lax.axis_index("c") * 16 + lax.axis_index("s")

        pltpu.sync_copy(pidx_hbm, pidx_b)
        pltpu.sync_copy(cg_hbm, cg_b)
        pltpu.sync_copy(tptr_hbm, tptr_b)

        @plsc.parallel_loop(0, _NNZ, _LANES)
        def fill_body(base):
            vpi = pidx_b[pl.ds(base, _LANES)]
            vcg = cg_b[pl.ds(base, _LANES)]
            for j in range(_LANES):
                pidx_s[base + j] = vpi[j]
                cg_s[base + j] = vcg[j]

        @plsc.parallel_loop(0, _TPTR_PAD, _LANES)
        def fill_tptr(base):
            vtp = tptr_b[pl.ds(base, _LANES)]
            for j in range(_LANES):
                tptr_s[base + j] = vtp[j]

        def batch_body(i, carry):
            b = wid * _BPW + i
            pltpu.sync_copy(x1_hbm.at[b], x1_v)
            pltpu.sync_copy(x2_hbm.at[b], x2_v)
            pltpu.sync_copy(w_hbm.at[b], w_v)

            zero = jnp.zeros((_LANES,), jnp.float32)

            @plsc.parallel_loop(0, _M, 1)
            def zero_body(m):
                for kk in range(_CCHUNKS):
                    out_v[m, pl.ds(kk * _LANES, _LANES)] = zero

            def path_body(t, carry2):
                st = tptr_s[t]
                en = tptr_s[t + 1]
                gw = [w_v[t, pl.ds(kk * _LANES, _LANES)]
                      for kk in range(_CCHUNKS)]

                @plsc.parallel_loop(st, en, 1, unroll=2)
                def e_body(n):
                    pidx = pidx_s[n]
                    cgs = cg_s[n]
                    o1 = pidx & 15
                    o2 = lax.shift_right_logical(pidx, 4) & 15
                    oo = lax.shift_right_logical(pidx, 8)
                    for kk in range(_CCHUNKS):
                        g1 = x1_v[o1, pl.ds(kk * _LANES, _LANES)]
                        g2 = x2_v[o2, pl.ds(kk * _LANES, _LANES)]
                        t_ = g1 * g2 * gw[kk] * cgs
                        plsc.addupdate(
                            out_v.at[oo, pl.ds(kk * _LANES, _LANES)], t_)
                return carry2
            lax.fori_loop(0, _NT, path_body, 0)

            pltpu.sync_copy(out_v, out_hbm.at[b])
            return carry
        lax.fori_loop(0, _BPW, batch_body, 0)

    return k(x1, x2, w, cg, pidx, tptr_pad)


def kernel(x1, x2, weight, CG_vals, l_ind_M1M2, M1, M2, M_ptr_M1M2):
    n_idx = jnp.arange(_NNZ, dtype=jnp.int32)
    seg = jnp.sum(n_idx[None, :] >= M_ptr_M1M2[1:_M, None],
                  axis=0, dtype=jnp.int32)
    perm = jnp.argsort(l_ind_M1M2, stable=True)
    pidx = (M1 | (M2 << 4) | (seg << 8))[perm]
    cgp = CG_vals[perm]
    hist = jnp.zeros((_NT,), jnp.int32).at[l_ind_M1M2].add(1)
    tptr = jnp.concatenate([jnp.zeros((1,), jnp.int32), jnp.cumsum(hist)])
    tptr_pad = jnp.concatenate(
        [tptr, jnp.zeros((_TPTR_PAD - _NT - 1,), jnp.int32)]).astype(jnp.int32)
    return _sc_tensor_product(x1, x2, weight, cgp, pidx, tptr_pad)


# speedup vs baseline: 1.5672x; 1.5672x over previous
import functools

import jax
import jax.numpy as jnp
from jax import lax
from jax.experimental import pallas as pl
from jax.experimental.pallas import tpu as pltpu
from jax.experimental.pallas import tpu_sc as plsc

_B = 1024
_M = 16
_C = 128
_NNZ = 512
_NT = 34

_LANES = 16
_NW = 32
_BPW = _B // _NW
_CCHUNKS = _C // _LANES
_MPTR_PAD = 32


def _sc_tensor_product(x1, x2, w, cg, p12, paw, mptr_pad):
    mesh = plsc.VectorSubcoreMesh(core_axis_name="c", subcore_axis_name="s")

    @functools.partial(
        pl.kernel,
        mesh=mesh,
        out_type=jax.ShapeDtypeStruct((_B, _M, _C), jnp.float32),
        compiler_params=pltpu.CompilerParams(needs_layout_passes=False),
        scratch_types=[
            pltpu.SMEM((_NNZ,), jnp.int32),
            pltpu.SMEM((_NNZ,), jnp.int32),
            pltpu.SMEM((_NNZ,), jnp.float32),
            pltpu.SMEM((_MPTR_PAD,), jnp.int32),
            pltpu.VMEM((_NNZ,), jnp.int32),
            pltpu.VMEM((_NNZ,), jnp.int32),
            pltpu.VMEM((_NNZ,), jnp.float32),
            pltpu.VMEM((_MPTR_PAD,), jnp.int32),
            pltpu.VMEM((_M, _C), jnp.float32),
            pltpu.VMEM((_M, _C), jnp.float32),
            pltpu.VMEM((_NT, _C), jnp.float32),
            pltpu.VMEM((_M, _C), jnp.float32),
            pltpu.VMEM((_M, _C), jnp.float32),
            pltpu.VMEM((_NT, _C), jnp.float32),
            pltpu.VMEM((_M, _C), jnp.float32),
            pltpu.SemaphoreType.DMA,
            pltpu.SemaphoreType.DMA,
        ],
    )
    def k(x1_hbm, x2_hbm, w_hbm, cg_hbm, p12_hbm, paw_hbm, mptr_hbm,
          out_hbm, p12_s, paw_s, cg_s, mptr_s, p12_b, paw_b, cg_b, mptr_b,
          x1_va, x2_va, w_va, x1_vb, x2_vb, w_vb, out_v, sem_a, sem_b):
        wid = lax.axis_index("c") * 16 + lax.axis_index("s")

        pltpu.sync_copy(p12_hbm, p12_b)
        pltpu.sync_copy(paw_hbm, paw_b)
        pltpu.sync_copy(cg_hbm, cg_b)
        pltpu.sync_copy(mptr_hbm, mptr_b)

        @plsc.parallel_loop(0, _NNZ, _LANES)
        def fill_body(base):
            v12 = p12_b[pl.ds(base, _LANES)]
            vaw = paw_b[pl.ds(base, _LANES)]
            vcg = cg_b[pl.ds(base, _LANES)]
            for j in range(_LANES):
                p12_s[base + j] = v12[j]
                paw_s[base + j] = vaw[j]
                cg_s[base + j] = vcg[j]

        @plsc.parallel_loop(0, _MPTR_PAD, _LANES)
        def fill_mptr(base):
            vmp = mptr_b[pl.ds(base, _LANES)]
            for j in range(_LANES):
                mptr_s[base + j] = vmp[j]

        b0 = wid * _BPW
        bufs = ((x1_va, x2_va, w_va, sem_a), (x1_vb, x2_vb, w_vb, sem_b))

        def start_copies(b, buf):
            x1d, x2d, wd, sem = buf
            pltpu.async_copy(x1_hbm.at[b], x1d, sem)
            pltpu.async_copy(x2_hbm.at[b], x2d, sem)
            pltpu.async_copy(w_hbm.at[b], wd, sem)

        def wait_copies(buf):
            x1d, x2d, wd, sem = buf
            pltpu.make_async_copy(x1_hbm.at[b0], x1d, sem).wait()
            pltpu.make_async_copy(x2_hbm.at[b0], x2d, sem).wait()
            pltpu.make_async_copy(w_hbm.at[b0], wd, sem).wait()

        def compute(b, buf):
            x1d, x2d, wd, _ = buf

            def seg_body(m, carry2):
                st = mptr_s[m]
                en = mptr_s[m + 1]
                zero = jnp.zeros((_LANES,), jnp.float32)
                init = (zero,) * _CCHUNKS

                @plsc.parallel_loop(st, en, 1, unroll=2, carry=init)
                def acc_fin(n, acc):
                    s12 = p12_s[n]
                    aws = paw_s[n]
                    cgs = cg_s[n]
                    o1 = s12 & 255
                    o2 = lax.shift_right_logical(s12, 8)
                    new = []
                    for kk in range(_CCHUNKS):
                        g1 = x1d[o1, pl.ds(kk * _LANES, _LANES)]
                        g2 = x2d[o2, pl.ds(kk * _LANES, _LANES)]
                        gw = wd[aws, pl.ds(kk * _LANES, _LANES)]
                        new.append(acc[kk] + g1 * g2 * gw * cgs)
                    return tuple(new)

                for kk in range(_CCHUNKS):
                    out_v[m, pl.ds(kk * _LANES, _LANES)] = acc_fin[kk]
                return carry2
            lax.fori_loop(0, _M, seg_body, 0)

            pltpu.sync_copy(out_v, out_hbm.at[b])

        start_copies(b0, bufs[0])

        def batch_pair(i2, carry):
            for par in range(2):
                i = i2 * 2 + par
                b = b0 + i
                buf = bufs[par]
                nxt = bufs[1 - par]
                wait_copies(buf)
                start_copies(b0 + jnp.minimum(i + 1, _BPW - 1), nxt)
                compute(b, buf)
            return carry
        lax.fori_loop(0, _BPW // 2, batch_pair, 0)
        wait_copies(bufs[0])

    return k(x1, x2, w, cg, p12, paw, mptr_pad)


def kernel(x1, x2, weight, CG_vals, l_ind_M1M2, M1, M2, M_ptr_M1M2):
    p12 = M1 | (M2 << 8)
    mptr_pad = jnp.concatenate(
        [M_ptr_M1M2, jnp.zeros((_MPTR_PAD - _M - 1,), jnp.int32)])
    return _sc_tensor_product(x1, x2, weight, CG_vals, p12, l_ind_M1M2,
                              mptr_pad)


# speedup vs baseline: 1.5900x; 1.0146x over previous
import functools

import jax
import jax.numpy as jnp
from jax import lax
from jax.experimental import pallas as pl
from jax.experimental.pallas import tpu as pltpu
from jax.experimental.pallas import tpu_sc as plsc

_B = 1024
_M = 16
_C = 128
_NNZ = 512
_NT = 34

_LANES = 16
_NW = 32
_BPW = _B // _NW
_CCHUNKS = _C // _LANES
_MPTR_PAD = 32


def _sc_tensor_product(x1, x2, w, cg, p12, paw, mptr_pad):
    mesh = plsc.VectorSubcoreMesh(core_axis_name="c", subcore_axis_name="s")

    @functools.partial(
        pl.kernel,
        mesh=mesh,
        out_type=jax.ShapeDtypeStruct((_B, _M, _C), jnp.float32),
        compiler_params=pltpu.CompilerParams(needs_layout_passes=False),
        scratch_types=[
            pltpu.SMEM((_NNZ,), jnp.int32),
            pltpu.SMEM((_NNZ,), jnp.int32),
            pltpu.SMEM((_NNZ,), jnp.float32),
            pltpu.SMEM((_MPTR_PAD,), jnp.int32),
            pltpu.VMEM((_NNZ,), jnp.int32),
            pltpu.VMEM((_NNZ,), jnp.int32),
            pltpu.VMEM((_NNZ,), jnp.float32),
            pltpu.VMEM((_MPTR_PAD,), jnp.int32),
            pltpu.VMEM((_M, _C), jnp.float32),
            pltpu.VMEM((_M, _C), jnp.float32),
            pltpu.VMEM((_NT, _C), jnp.float32),
            pltpu.VMEM((_M, _C), jnp.float32),
            pltpu.VMEM((_M, _C), jnp.float32),
            pltpu.VMEM((_NT, _C), jnp.float32),
            pltpu.VMEM((_M, _C), jnp.float32),
            pltpu.VMEM((_M, _C), jnp.float32),
            pltpu.SemaphoreType.DMA,
            pltpu.SemaphoreType.DMA,
            pltpu.SemaphoreType.DMA,
            pltpu.SemaphoreType.DMA,
        ],
    )
    def k(x1_hbm, x2_hbm, w_hbm, cg_hbm, p12_hbm, paw_hbm, mptr_hbm,
          out_hbm, p12_s, paw_s, cg_s, mptr_s, p12_b, paw_b, cg_b, mptr_b,
          x1_va, x2_va, w_va, x1_vb, x2_vb, w_vb, out_va, out_vb,
          sem_a, sem_b, sem_oa, sem_ob):
        wid = lax.axis_index("c") * 16 + lax.axis_index("s")

        pltpu.sync_copy(p12_hbm, p12_b)
        pltpu.sync_copy(paw_hbm, paw_b)
        pltpu.sync_copy(cg_hbm, cg_b)
        pltpu.sync_copy(mptr_hbm, mptr_b)

        @plsc.parallel_loop(0, _NNZ, _LANES)
        def fill_body(base):
            v12 = p12_b[pl.ds(base, _LANES)]
            vaw = paw_b[pl.ds(base, _LANES)]
            vcg = cg_b[pl.ds(base, _LANES)]
            for j in range(_LANES):
                p12_s[base + j] = v12[j]
                paw_s[base + j] = vaw[j]
                cg_s[base + j] = vcg[j]

        @plsc.parallel_loop(0, _MPTR_PAD, _LANES)
        def fill_mptr(base):
            vmp = mptr_b[pl.ds(base, _LANES)]
            for j in range(_LANES):
                mptr_s[base + j] = vmp[j]

        b0 = wid * _BPW
        bufs = ((x1_va, x2_va, w_va, sem_a), (x1_vb, x2_vb, w_vb, sem_b))
        obufs = ((out_va, sem_oa), (out_vb, sem_ob))

        def start_copies(b, buf):
            x1d, x2d, wd, sem = buf
            pltpu.async_copy(x1_hbm.at[b], x1d, sem)
            pltpu.async_copy(x2_hbm.at[b], x2d, sem)
            pltpu.async_copy(w_hbm.at[b], wd, sem)

        def wait_copies(buf):
            x1d, x2d, wd, sem = buf
            pltpu.make_async_copy(x1_hbm.at[b0], x1d, sem).wait()
            pltpu.make_async_copy(x2_hbm.at[b0], x2d, sem).wait()
            pltpu.make_async_copy(w_hbm.at[b0], wd, sem).wait()

        def compute(b, i2, buf, obuf):
            x1d, x2d, wd, _ = buf
            out_v, sem_o = obuf

            @pl.when(i2 > 0)
            def _():
                pltpu.make_async_copy(out_v, out_hbm.at[b0], sem_o).wait()

            def seg_body(m, carry2):
                st = mptr_s[m]
                en = mptr_s[m + 1]
                zero = jnp.zeros((_LANES,), jnp.float32)
                init = (zero,) * _CCHUNKS

                @plsc.parallel_loop(st, en, 1, unroll=2, carry=init)
                def acc_fin(n, acc):
                    s12 = p12_s[n]
                    aws = paw_s[n]
                    cgs = cg_s[n]
                    o1 = s12 & 255
                    o2 = lax.shift_right_logical(s12, 8)
                    new = []
                    for kk in range(_CCHUNKS):
                        g1 = x1d[o1, pl.ds(kk * _LANES, _LANES)]
                        g2 = x2d[o2, pl.ds(kk * _LANES, _LANES)]
                        gw = wd[aws, pl.ds(kk * _LANES, _LANES)]
                        new.append(acc[kk] + g1 * g2 * gw * cgs)
                    return tuple(new)

                for kk in range(_CCHUNKS):
                    out_v[m, pl.ds(kk * _LANES, _LANES)] = acc_fin[kk]
                return carry2
            lax.fori_loop(0, _M, seg_body, 0)

            pltpu.async_copy(out_v, out_hbm.at[b], sem_o)

        start_copies(b0, bufs[0])

        def batch_pair(i2, carry):
            for par in range(2):
                i = i2 * 2 + par
                b = b0 + i
                buf = bufs[par]
                nxt = bufs[1 - par]
                wait_copies(buf)
                start_copies(b0 + jnp.minimum(i + 1, _BPW - 1), nxt)
                compute(b, i2, buf, obufs[par])
            return carry
        lax.fori_loop(0, _BPW // 2, batch_pair, 0)
        wait_copies(bufs[0])
        for out_v, sem_o in obufs:
            pltpu.make_async_copy(out_v, out_hbm.at[b0], sem_o).wait()

    return k(x1, x2, w, cg, p12, paw, mptr_pad)


def kernel(x1, x2, weight, CG_vals, l_ind_M1M2, M1, M2, M_ptr_M1M2):
    p12 = M1 | (M2 << 8)
    mptr_pad = jnp.concatenate(
        [M_ptr_M1M2, jnp.zeros((_MPTR_PAD - _M - 1,), jnp.int32)])
    return _sc_tensor_product(x1, x2, weight, CG_vals, p12, l_ind_M1M2,
                              mptr_pad)
